# baseline (speedup 1.0000x reference)
import functools
import math

import jax
import jax.numpy as jnp
from jax import lax
from jax.experimental import pallas as pl
from jax.experimental.pallas import tpu as pltpu
from jax.experimental.pallas import tpu_sc as plsc

D_MODEL = 64
SCALE = math.sqrt(D_MODEL)
LANES = 16

_info = plsc.get_sparse_core_info()
NC, NS = _info.num_cores, _info.num_subcores
NW = NC * NS


def _emb_body(n_rows, chunk, table_hbm, idx_hbm, out_hbm,
              idx_v, g0, g1, s0, s1, gsem0, gsem1, ssem0, ssem1):
    wid = lax.axis_index("s") * NC + lax.axis_index("c")
    base = wid * n_rows
    n_chunks = n_rows // chunk
    gbuf, sbuf = (g0, g1), (s0, s1)
    gsem, ssem = (gsem0, gsem1), (ssem0, ssem1)

    pltpu.sync_copy(idx_hbm.at[pl.ds(base, n_rows)], idx_v)

    def start_gather(i, b):
        pltpu.async_copy(
            table_hbm.at[idx_v.at[pl.ds(i * chunk, chunk)]], gbuf[b], gsem[b])

    def wait_gather(b):
        pltpu.make_async_copy(
            table_hbm.at[idx_v.at[pl.ds(0, chunk)]], gbuf[b], gsem[b]).wait()

    def start_store(i, b):
        pltpu.async_copy(
            sbuf[b], out_hbm.at[pl.ds(base + i * chunk, chunk)], ssem[b])

    def wait_store(b):
        pltpu.make_async_copy(
            sbuf[b], out_hbm.at[pl.ds(base, chunk)], ssem[b]).wait()

    start_gather(0, 0)
    start_gather(1, 1)

    def do_pair(step, carry):
        for b in (0, 1):
            i = step * 2 + b
            wait_gather(b)

            @pl.when(i >= 2)
            def _():
                wait_store(b)

            def scale_row(r, c):
                for l in range(D_MODEL // LANES):
                    s = pl.ds(l * LANES, LANES)
                    sbuf[b][r, s] = gbuf[b][r, s] * SCALE
                return c

            lax.fori_loop(0, chunk, scale_row, 0, unroll=4)
            start_store(i, b)

            @pl.when(i + 2 < n_chunks)
            def _():
                start_gather(i + 2, b)
        return carry

    lax.fori_loop(0, n_chunks // 2, do_pair, 0)
    wait_store(0)
    wait_store(1)


def kernel(x, lut):
    b, s = x.shape
    n = b * s
    idx = x.reshape(n).astype(jnp.int32)
    n_rows = n // NW
    chunk = 400

    body = functools.partial(_emb_body, n_rows, chunk)
    out = pl.kernel(
        body,
        out_type=jax.ShapeDtypeStruct((n, D_MODEL), jnp.float32),
        mesh=plsc.VectorSubcoreMesh(core_axis_name="c", subcore_axis_name="s"),
        compiler_params=pltpu.CompilerParams(use_tc_tiling_on_sc=False),
        scratch_types=[
            pltpu.VMEM((n_rows,), jnp.int32),
            pltpu.VMEM((chunk, D_MODEL), jnp.float32),
            pltpu.VMEM((chunk, D_MODEL), jnp.float32),
            pltpu.VMEM((chunk, D_MODEL), jnp.float32),
            pltpu.VMEM((chunk, D_MODEL), jnp.float32),
            pltpu.SemaphoreType.DMA,
            pltpu.SemaphoreType.DMA,
            pltpu.SemaphoreType.DMA,
            pltpu.SemaphoreType.DMA,
        ],
    )(lut, idx)
    return out.reshape(b, s, D_MODEL)
